# baseline (speedup 1.0000x reference)
import functools

import jax
import jax.numpy as jnp
from jax import lax
from jax.experimental import pallas as pl
from jax.experimental.pallas import tpu as pltpu
from jax.experimental.pallas import tpu_sc as plsc


@functools.lru_cache(maxsize=None)
def _build_pos_kernel(d, em, h, w):
    info = plsc.get_sparse_core_info()
    NC, NS, L = info.num_cores, info.num_subcores, info.num_lanes
    NW = NC * NS
    f2 = 2 * em
    NG = NW // 2
    YPG = h // NG
    DPW = d // 2
    TX, TC = w // 8, f2 // 128
    NT = TX * TC
    KL = 128 // L
    assert h % NG == 0 and d % 2 == 0 and em % 128 == 0 and w % 8 == 0
    mesh = plsc.VectorSubcoreMesh(core_axis_name="c", subcore_axis_name="s")

    @functools.partial(
        pl.kernel,
        mesh=mesh,
        compiler_params=pltpu.CompilerParams(needs_layout_passes=False),
        out_type=jax.ShapeDtypeStruct((d, h, NT, 8, 128), jnp.float32),
        scratch_types=[
            pltpu.VMEM((w, em), jnp.float32),
            pltpu.VMEM((h, em), jnp.float32),
            pltpu.VMEM((YPG, NT, 8, 128), jnp.float32),
            pltpu.SemaphoreType.DMA,
        ],
    )
    def pos_kernel(col_hbm, row_hbm, out_hbm, col_v, row_v, stripe_v, sem):
        wid = lax.axis_index("s") * NC + lax.axis_index("c")
        g = wid // 2
        half = wid % 2
        d0 = half * DPW
        pltpu.sync_copy(col_hbm.at[pl.ds(0, w)], col_v)
        pltpu.sync_copy(row_hbm.at[pl.ds(0, h)], row_v)

        def build_row(yy):
            def txbody(tx, carry):
                def sxbody(sx, carry2):
                    x = 8 * tx + sx
                    for tc in range(TC // 2):
                        for k in range(KL):
                            stripe_v[yy, TC * tx + tc, sx, pl.ds(k * L, L)] = (
                                col_v[x, pl.ds(128 * tc + k * L, L)])
                    for tc in range(TC // 2):
                        for k in range(KL):
                            stripe_v[yy, TC * tx + TC // 2 + tc, sx,
                                     pl.ds(k * L, L)] = (
                                row_v[g * YPG + yy,
                                      pl.ds(128 * tc + k * L, L)])
                    return carry2
                return lax.fori_loop(0, 8, sxbody, carry)
            lax.fori_loop(0, TX, txbody, 0)

        def fire_row(yy):
            return [
                pltpu.async_copy(
                    stripe_v.at[yy],
                    out_hbm.at[d0 + dd, g * YPG + yy],
                    sem)
                for dd in range(DPW)
            ]

        build_row(0)
        prev = fire_row(0)
        for yy in range(1, YPG):
            build_row(yy)
            for hd in prev:
                hd.wait()
            prev = fire_row(yy)
        for hd in prev:
            hd.wait()

    return pos_kernel


def kernel(scan, row_embed, col_embed, dep_embed):
    d, em, h, w = scan.shape
    f2 = 2 * em
    out = _build_pos_kernel(d, em, h, w)(col_embed, row_embed)
    return (out.reshape(d, h, w // 8, f2 // 128, 8, 128)
            .transpose(0, 3, 5, 1, 2, 4)
            .reshape(d, f2, h, w))


# speedup vs baseline: 1.0487x; 1.0487x over previous
import functools

import jax
import jax.numpy as jnp
from jax import lax
from jax.experimental import pallas as pl
from jax.experimental.pallas import tpu as pltpu
from jax.experimental.pallas import tpu_sc as plsc


@functools.lru_cache(maxsize=None)
def _build_pos_kernel(d, em, h, w):
    info = plsc.get_sparse_core_info()
    NC, NS, L = info.num_cores, info.num_subcores, info.num_lanes
    NW = NC * NS
    f2 = 2 * em
    NG = NW // 2
    YPG = h // NG
    DPW = d // 2
    assert h % NG == 0 and d % 2 == 0 and em % L == 0
    mesh = plsc.VectorSubcoreMesh(core_axis_name="c", subcore_axis_name="s")

    @functools.partial(
        pl.kernel,
        mesh=mesh,
        compiler_params=pltpu.CompilerParams(needs_layout_passes=False),
        out_type=jax.ShapeDtypeStruct((d, h, w, f2), jnp.float32),
        scratch_types=[
            pltpu.VMEM((w, em), jnp.float32),
            pltpu.VMEM((h, em), jnp.float32),
            pltpu.VMEM((YPG, w, f2), jnp.float32),
            pltpu.SemaphoreType.DMA,
        ],
    )
    def pos_kernel(col_hbm, row_hbm, out_hbm, col_v, row_v, stripe_v, sem):
        wid = lax.axis_index("s") * NC + lax.axis_index("c")
        g = wid // 2
        half = wid % 2
        pltpu.sync_copy(col_hbm.at[pl.ds(0, w)], col_v)
        pltpu.sync_copy(row_hbm.at[pl.ds(0, h)], row_v)

        rvs = [[row_v[g * YPG + yy, pl.ds(k * L, L)] for k in range(em // L)]
               for yy in range(YPG)]

        def xbody(x, carry):
            for k in range(em // L):
                v = col_v[x, pl.ds(k * L, L)]
                for yy in range(YPG):
                    stripe_v[yy, x, pl.ds(k * L, L)] = v
            for yy in range(YPG):
                for k in range(em // L):
                    stripe_v[yy, x, pl.ds(em + k * L, L)] = rvs[yy][k]
            return carry

        lax.fori_loop(0, w, xbody, 0)

        y0 = g * YPG
        handles = [
            pltpu.async_copy(
                stripe_v,
                out_hbm.at[half * DPW + dd, pl.ds(y0, YPG)],
                sem)
            for dd in range(DPW)
        ]
        for hd in handles:
            hd.wait()

    return pos_kernel


def kernel(scan, row_embed, col_embed, dep_embed):
    d, em, h, w = scan.shape
    out = _build_pos_kernel(d, em, h, w)(col_embed, row_embed)
    return out.transpose(0, 3, 1, 2)
